# Initial kernel scaffold; baseline (speedup 1.0000x reference)
import jax
import jax.numpy as jnp
from jax.experimental import pallas as pl


def kernel(x, W1, b1, W2, b2):
    raise NotImplementedError("write your pallas kernel here")


# speedup vs baseline: 6.7472x; 6.7472x over previous
import jax
import jax.numpy as jnp
from jax.experimental import pallas as pl
from jax.experimental.pallas import tpu as pltpu

N_EXPERTS = 64
TOPK = 8
BLK = 1024


def _router_kernel(x_ref, w1_ref, w2_ref, b1_ref, b2_ref, out_ref, idx_ref):
    x = x_ref[...].astype(jnp.bfloat16)
    l1 = jnp.dot(x, w1_ref[...].astype(jnp.bfloat16),
                 preferred_element_type=jnp.float32) + b1_ref[...]
    l2 = jnp.dot(x, w2_ref[...].astype(jnp.bfloat16),
                 preferred_element_type=jnp.float32) + b2_ref[...]
    logits = l1 + l2
    rows = logits.shape[0]
    lane = jax.lax.broadcasted_iota(jnp.int32, (rows, N_EXPERTS), 1)
    k_lane = jax.lax.broadcasted_iota(jnp.int32, (rows, TOPK), 1)
    v = logits
    sel = jnp.zeros((rows, N_EXPERTS), jnp.bool_)
    idx_out = jnp.zeros((rows, TOPK), jnp.int32)
    for k in range(TOPK):
        m = jnp.max(v, axis=1, keepdims=True)
        idx = jnp.min(jnp.where(v == m, lane, N_EXPERTS), axis=1, keepdims=True)
        chosen = lane == idx
        sel = jnp.logical_or(sel, chosen)
        idx_out = jnp.where(k_lane == k, idx, idx_out)
        v = jnp.where(chosen, -jnp.inf, v)
    m1 = jnp.max(logits, axis=1, keepdims=True)
    p = jnp.where(sel, jnp.exp(logits - m1), 0.0)
    out_ref[...] = p / jnp.sum(p, axis=1, keepdims=True)
    idx_ref[...] = idx_out


@jax.jit
def kernel(x, W1, b1, W2, b2):
    B, S, E = x.shape
    T = B * S
    xf = x.reshape(T, E)
    b1r = b1.reshape(1, N_EXPERTS)
    b2r = b2.reshape(1, N_EXPERTS)
    router, idx = pl.pallas_call(
        _router_kernel,
        grid=(T // BLK,),
        in_specs=[
            pl.BlockSpec((BLK, E), lambda i: (i, 0)),
            pl.BlockSpec((E, N_EXPERTS), lambda i: (0, 0)),
            pl.BlockSpec((E, N_EXPERTS), lambda i: (0, 0)),
            pl.BlockSpec((1, N_EXPERTS), lambda i: (0, 0)),
            pl.BlockSpec((1, N_EXPERTS), lambda i: (0, 0)),
        ],
        out_specs=[
            pl.BlockSpec((BLK, N_EXPERTS), lambda i: (i, 0)),
            pl.BlockSpec((BLK, TOPK), lambda i: (i, 0)),
        ],
        out_shape=[
            jax.ShapeDtypeStruct((T, N_EXPERTS), jnp.float32),
            jax.ShapeDtypeStruct((T, TOPK), jnp.int32),
        ],
        compiler_params=pltpu.CompilerParams(
            dimension_semantics=("parallel",),
        ),
    )(xf, W1, W2, b1r, b2r)
    return router.reshape(B, S, N_EXPERTS), idx.reshape(B, S, TOPK)


# speedup vs baseline: 12.2565x; 1.8165x over previous
import jax
import jax.numpy as jnp
from jax.experimental import pallas as pl
from jax.experimental.pallas import tpu as pltpu

N_EXPERTS = 64
TOPK = 8
BLK = 1024


def _router_kernel(x_ref, w1t_ref, w2t_ref, b1_ref, b2_ref, out_ref, idx_ref):
    x = x_ref[...].astype(jnp.bfloat16)
    cdims = (((1,), (1,)), ((), ()))
    l1 = jax.lax.dot_general(w1t_ref[...].astype(jnp.bfloat16), x, cdims,
                             preferred_element_type=jnp.float32) + b1_ref[...]
    l2 = jax.lax.dot_general(w2t_ref[...].astype(jnp.bfloat16), x, cdims,
                             preferred_element_type=jnp.float32) + b2_ref[...]
    logits = l1 + l2
    cols = logits.shape[1]
    ids = jax.lax.broadcasted_iota(jnp.int32, (N_EXPERTS, cols), 0).astype(
        jnp.float32)
    krow = jax.lax.broadcasted_iota(jnp.int32, (TOPK, cols), 0).astype(
        jnp.float32)
    v = logits
    idx_out = jnp.zeros((TOPK, cols), jnp.float32)
    m1 = None
    for k in range(TOPK):
        m = jnp.max(v, axis=0, keepdims=True)
        if k == 0:
            m1 = m
        idx = jnp.min(jnp.where(v == m, ids, float(N_EXPERTS)), axis=0,
                      keepdims=True)
        idx_out = jnp.where(krow == float(k), idx, idx_out)
        v = jnp.where(ids == idx, -jnp.inf, v)
    p = jnp.where(v == -jnp.inf, jnp.exp(logits - m1), 0.0)
    p = p / jnp.sum(p, axis=0, keepdims=True)
    out_ref[...] = p.T
    idx_ref[...] = idx_out.astype(jnp.int32).T


@jax.jit
def kernel(x, W1, b1, W2, b2):
    B, S, E = x.shape
    T = B * S
    xf = x.reshape(T, E)
    w1t = W1.T
    w2t = W2.T
    b1r = b1.reshape(N_EXPERTS, 1)
    b2r = b2.reshape(N_EXPERTS, 1)
    router, idx = pl.pallas_call(
        _router_kernel,
        grid=(T // BLK,),
        in_specs=[
            pl.BlockSpec((BLK, E), lambda i: (i, 0)),
            pl.BlockSpec((N_EXPERTS, E), lambda i: (0, 0)),
            pl.BlockSpec((N_EXPERTS, E), lambda i: (0, 0)),
            pl.BlockSpec((N_EXPERTS, 1), lambda i: (0, 0)),
            pl.BlockSpec((N_EXPERTS, 1), lambda i: (0, 0)),
        ],
        out_specs=[
            pl.BlockSpec((BLK, N_EXPERTS), lambda i: (i, 0)),
            pl.BlockSpec((BLK, TOPK), lambda i: (i, 0)),
        ],
        out_shape=[
            jax.ShapeDtypeStruct((T, N_EXPERTS), jnp.float32),
            jax.ShapeDtypeStruct((T, TOPK), jnp.int32),
        ],
        compiler_params=pltpu.CompilerParams(
            dimension_semantics=("parallel",),
        ),
    )(xf, w1t, w2t, b1r, b2r)
    return router.reshape(B, S, N_EXPERTS), idx.reshape(B, S, TOPK)


# speedup vs baseline: 13.2695x; 1.0826x over previous
import jax
import jax.numpy as jnp
from jax.experimental import pallas as pl
from jax.experimental.pallas import tpu as pltpu

N_EXPERTS = 64
TOPK = 8
BLK = 1024


def _router_kernel(x_ref, wcat_ref, b1_ref, b2_ref, out_ref, idx_ref):
    xb = x_ref[0].astype(jnp.bfloat16)
    ll = jax.lax.dot_general(wcat_ref[...].astype(jnp.bfloat16), xb,
                             (((0,), (1,)), ((), ())),
                             preferred_element_type=jnp.float32)
    l1 = ll[:N_EXPERTS] + b1_ref[...]
    l2 = ll[N_EXPERTS:] + b2_ref[...]
    logits = l1 + l2
    cols = logits.shape[1]
    ids = jax.lax.broadcasted_iota(jnp.int32, (N_EXPERTS, cols), 0).astype(
        jnp.float32)
    v = logits
    idx_rows = []
    m1 = None
    for k in range(TOPK):
        m = jnp.max(v, axis=0, keepdims=True)
        if k == 0:
            m1 = m
        eq = v == m
        idx = jnp.min(jnp.where(eq, ids, float(N_EXPERTS)), axis=0,
                      keepdims=True)
        idx_rows.append(idx)
        v = jnp.where(ids == idx, -jnp.inf, v)
    idx_out = jnp.concatenate(idx_rows, axis=0)
    p = jnp.where(v == -jnp.inf, jnp.exp(logits - m1), 0.0)
    p = p / jnp.sum(p, axis=0, keepdims=True)
    out_ref[...] = p.T[None]
    idx_ref[...] = idx_out.astype(jnp.int32).T[None]


@jax.jit
def kernel(x, W1, b1, W2, b2):
    B, S, E = x.shape
    wcat = jnp.concatenate([W1, W2], axis=1)
    b1r = b1.reshape(N_EXPERTS, 1)
    b2r = b2.reshape(N_EXPERTS, 1)
    router, idx = pl.pallas_call(
        _router_kernel,
        grid=(B, S // BLK),
        in_specs=[
            pl.BlockSpec((1, BLK, E), lambda b, i: (b, i, 0)),
            pl.BlockSpec((E, 2 * N_EXPERTS), lambda b, i: (0, 0)),
            pl.BlockSpec((N_EXPERTS, 1), lambda b, i: (0, 0)),
            pl.BlockSpec((N_EXPERTS, 1), lambda b, i: (0, 0)),
        ],
        out_specs=[
            pl.BlockSpec((1, BLK, N_EXPERTS), lambda b, i: (b, i, 0)),
            pl.BlockSpec((1, BLK, TOPK), lambda b, i: (b, i, 0)),
        ],
        out_shape=[
            jax.ShapeDtypeStruct((B, S, N_EXPERTS), jnp.float32),
            jax.ShapeDtypeStruct((B, S, TOPK), jnp.int32),
        ],
        compiler_params=pltpu.CompilerParams(
            dimension_semantics=("parallel", "parallel"),
        ),
    )(x, wcat, b1r, b2r)
    return router, idx


# speedup vs baseline: 14.9792x; 1.1288x over previous
import jax
import jax.numpy as jnp
from jax.experimental import pallas as pl
from jax.experimental.pallas import tpu as pltpu

N_EXPERTS = 64
TOPK = 8
NSPLIT = 4
QBLK = 512
BLK = NSPLIT * QBLK


def _router_kernel(x0_ref, x1_ref, x2_ref, x3_ref, wcat_ref, b1_ref, b2_ref,
                   out_ref, idx_ref):
    wb = wcat_ref[...].astype(jnp.bfloat16)
    parts = []
    for xr in (x0_ref, x1_ref, x2_ref, x3_ref):
        parts.append(jax.lax.dot_general(wb, xr[0].astype(jnp.bfloat16),
                                         (((0,), (1,)), ((), ())),
                                         preferred_element_type=jnp.float32))
    ll = jnp.concatenate(parts, axis=1)
    l1 = ll[:N_EXPERTS] + b1_ref[...]
    l2 = ll[N_EXPERTS:] + b2_ref[...]
    logits = l1 + l2
    cols = logits.shape[1]
    ids = jax.lax.broadcasted_iota(jnp.int32, (N_EXPERTS, cols), 0).astype(
        jnp.float32)
    v = logits
    idx_rows = []
    m1 = None
    for k in range(TOPK):
        m = jnp.max(v, axis=0, keepdims=True)
        if k == 0:
            m1 = m
        eq = v == m
        idx = jnp.min(jnp.where(eq, ids, float(N_EXPERTS)), axis=0,
                      keepdims=True)
        idx_rows.append(idx)
        v = jnp.where(ids == idx, -jnp.inf, v)
    idx_out = jnp.concatenate(idx_rows, axis=0)
    p = jnp.where(v == -jnp.inf, jnp.exp(logits - m1), 0.0)
    p = p / jnp.sum(p, axis=0, keepdims=True)
    out_ref[...] = p.T[None]
    idx_ref[...] = idx_out.astype(jnp.int32).T[None]


@jax.jit
def kernel(x, W1, b1, W2, b2):
    B, S, E = x.shape
    wcat = jnp.concatenate([W1, W2], axis=1)
    b1r = b1.reshape(N_EXPERTS, 1)
    b2r = b2.reshape(N_EXPERTS, 1)

    def xspec(q):
        return pl.BlockSpec((1, QBLK, E),
                            lambda b, i, q=q: (b, NSPLIT * i + q, 0))

    router, idx = pl.pallas_call(
        _router_kernel,
        grid=(B, S // BLK),
        in_specs=[
            xspec(0), xspec(1), xspec(2), xspec(3),
            pl.BlockSpec((E, 2 * N_EXPERTS), lambda b, i: (0, 0)),
            pl.BlockSpec((N_EXPERTS, 1), lambda b, i: (0, 0)),
            pl.BlockSpec((N_EXPERTS, 1), lambda b, i: (0, 0)),
        ],
        out_specs=[
            pl.BlockSpec((1, BLK, N_EXPERTS), lambda b, i: (b, i, 0)),
            pl.BlockSpec((1, BLK, TOPK), lambda b, i: (b, i, 0)),
        ],
        out_shape=[
            jax.ShapeDtypeStruct((B, S, N_EXPERTS), jnp.float32),
            jax.ShapeDtypeStruct((B, S, TOPK), jnp.int32),
        ],
        compiler_params=pltpu.CompilerParams(
            dimension_semantics=("parallel", "parallel"),
        ),
    )(x, x, x, x, wcat, b1r, b2r)
    return router, idx


# speedup vs baseline: 15.7025x; 1.0483x over previous
import jax
import jax.numpy as jnp
from jax.experimental import pallas as pl
from jax.experimental.pallas import tpu as pltpu

N_EXPERTS = 64
TOPK = 8
NSPLIT = 8
QBLK = 512
BLK = NSPLIT * QBLK


def _router_kernel(*refs):
    (x_refs, (wcat_ref, b1_ref, b2_ref, out_ref, idx_ref)) = refs[:NSPLIT], refs[NSPLIT:]
    wb = wcat_ref[...].astype(jnp.bfloat16)
    parts = []
    for xr in x_refs:
        parts.append(jax.lax.dot_general(wb, xr[0].astype(jnp.bfloat16),
                                         (((0,), (1,)), ((), ())),
                                         preferred_element_type=jnp.float32))
    ll = jnp.concatenate(parts, axis=1)
    l1 = ll[:N_EXPERTS] + b1_ref[...]
    l2 = ll[N_EXPERTS:] + b2_ref[...]
    logits = l1 + l2
    cols = logits.shape[1]
    ids = jax.lax.broadcasted_iota(jnp.int32, (N_EXPERTS, cols), 0).astype(
        jnp.float32)
    v = logits
    idx_rows = []
    m1 = None
    for k in range(TOPK):
        m = jnp.max(v, axis=0, keepdims=True)
        if k == 0:
            m1 = m
        eq = v == m
        idx = jnp.min(jnp.where(eq, ids, float(N_EXPERTS)), axis=0,
                      keepdims=True)
        idx_rows.append(idx)
        v = jnp.where(ids == idx, -jnp.inf, v)
    idx_out = jnp.concatenate(idx_rows, axis=0)
    p = jnp.where(v == -jnp.inf, jnp.exp(logits - m1), 0.0)
    p = p / jnp.sum(p, axis=0, keepdims=True)
    out_ref[...] = p.T[None]
    idx_ref[...] = idx_out.astype(jnp.int32).T[None]


@jax.jit
def kernel(x, W1, b1, W2, b2):
    B, S, E = x.shape
    wcat = jnp.concatenate([W1, W2], axis=1)
    b1r = b1.reshape(N_EXPERTS, 1)
    b2r = b2.reshape(N_EXPERTS, 1)

    def xspec(q):
        return pl.BlockSpec((1, QBLK, E),
                            lambda b, i, q=q: (b, NSPLIT * i + q, 0))

    router, idx = pl.pallas_call(
        _router_kernel,
        grid=(B, S // BLK),
        in_specs=[
            *[xspec(q) for q in range(NSPLIT)],
            pl.BlockSpec((E, 2 * N_EXPERTS), lambda b, i: (0, 0)),
            pl.BlockSpec((N_EXPERTS, 1), lambda b, i: (0, 0)),
            pl.BlockSpec((N_EXPERTS, 1), lambda b, i: (0, 0)),
        ],
        out_specs=[
            pl.BlockSpec((1, BLK, N_EXPERTS), lambda b, i: (b, i, 0)),
            pl.BlockSpec((1, BLK, TOPK), lambda b, i: (b, i, 0)),
        ],
        out_shape=[
            jax.ShapeDtypeStruct((B, S, N_EXPERTS), jnp.float32),
            jax.ShapeDtypeStruct((B, S, TOPK), jnp.int32),
        ],
        compiler_params=pltpu.CompilerParams(
            dimension_semantics=("parallel", "parallel"),
        ),
    )(*([x] * NSPLIT), wcat, b1r, b2r)
    return router, idx
